# Initial kernel scaffold; baseline (speedup 1.0000x reference)
import jax
import jax.numpy as jnp
from jax.experimental import pallas as pl


def kernel(inputs, embed_tables, wide_tables, w_dense, wide_b, W0, b0, W1, b1, W2, b2, W3, b3):
    raise NotImplementedError("write your pallas kernel here")


# speedup vs baseline: 6.8915x; 6.8915x over previous
import functools
import jax
import jax.numpy as jnp
from jax import lax
from jax.experimental import pallas as pl
from jax.experimental.pallas import tpu as pltpu
from jax.experimental.pallas import tpu_sc as plsc

N_DENSE = 13
N_SPARSE = 26
VOCAB = 100000
EMBED_DIM = 16
BATCH = 16384

NC = 2
NS = 16
LANES = 16
NW = NC * NS
B_PER_W = BATCH // NW
CH = 128
NCHUNK = B_PER_W // CH
ROWS = CH * N_SPARSE
SEG = 128
NSEG = ROWS // SEG
NSEG_PAD = 32


def _sc_gather(emb_flat, wide_flat, idx2d):
  mesh = plsc.VectorSubcoreMesh(core_axis_name="c", subcore_axis_name="s")

  @functools.partial(
      pl.kernel,
      mesh=mesh,
      compiler_params=pltpu.CompilerParams(use_tc_tiling_on_sc=False),
      out_type=[
          jax.ShapeDtypeStruct((BATCH * N_SPARSE, EMBED_DIM), jnp.float32),
          jax.ShapeDtypeStruct((BATCH * N_SPARSE,), jnp.float32),
      ],
      scratch_types=[
          pltpu.VMEM((NSEG_PAD, SEG), jnp.int32),
          pltpu.VMEM((ROWS, EMBED_DIM), jnp.float32),
          pltpu.VMEM((ROWS,), jnp.float32),
          pltpu.SemaphoreType.DMA,
          pltpu.SemaphoreType.DMA,
      ],
  )
  def k(emb_hbm, wide_hbm, idx_hbm, h_out, wval_out,
        idx_v, emb_v, wval_v, sem_e, sem_w):
    wid = lax.axis_index("s") * NC + lax.axis_index("c")

    def chunk_body(c, carry):
      seg0 = (wid * NCHUNK + c) * NSEG_PAD
      pltpu.sync_copy(idx_hbm.at[pl.ds(seg0, NSEG_PAD)], idx_v)
      cps = []
      for j in range(NSEG):
        cps.append(pltpu.async_copy(
            emb_hbm.at[idx_v.at[j]], emb_v.at[pl.ds(j * SEG, SEG)], sem_e))
        cps.append(pltpu.async_copy(
            wide_hbm.at[idx_v.at[j]], wval_v.at[pl.ds(j * SEG, SEG)], sem_w))
      for cp in cps:
        cp.wait()
      base = (wid * NCHUNK + c) * ROWS
      pltpu.sync_copy(emb_v, h_out.at[pl.ds(base, ROWS)])
      pltpu.sync_copy(wval_v, wval_out.at[pl.ds(base, ROWS)])
      return carry

    lax.fori_loop(0, NCHUNK, chunk_body, 0)

  return k(emb_flat, wide_flat, idx2d)


def _tc_mlp(h, xin, wval, wd_pad, wb, W0, b0, W1, b1, W2, b2, w3r, b3):
  BLK = 2048
  grid = (BATCH // BLK,)

  def body(h_ref, x_ref, wv_ref, wd_ref, wb_ref,
           W0_ref, b0_ref, W1_ref, b1_ref, W2_ref, b2_ref, w3_ref, b3_ref,
           o_ref):
    a = jnp.dot(h_ref[...], W0_ref[...], preferred_element_type=jnp.float32)
    a = jnp.maximum(a + b0_ref[...], 0.0)
    a = jnp.dot(a, W1_ref[...], preferred_element_type=jnp.float32)
    a = jnp.maximum(a + b1_ref[...], 0.0)
    a = jnp.dot(a, W2_ref[...], preferred_element_type=jnp.float32)
    a = jnp.maximum(a + b2_ref[...], 0.0)
    deep = jnp.sum(a * w3_ref[...], axis=1, keepdims=True) + b3_ref[...]
    dense = jnp.sum(x_ref[...] * wd_ref[...], axis=1, keepdims=True)
    wsum = jnp.sum(wv_ref[...], axis=1, keepdims=True)
    wide = dense + wsum + wb_ref[...]
    o_ref[...] = jax.nn.sigmoid(0.5 * (wide + deep))

  full = lambda shape: pl.BlockSpec(shape, lambda i: (0, 0))
  return pl.pallas_call(
      body,
      grid=grid,
      in_specs=[
          pl.BlockSpec((BLK, N_SPARSE * EMBED_DIM), lambda i: (i, 0)),
          pl.BlockSpec((BLK, N_DENSE + N_SPARSE), lambda i: (i, 0)),
          pl.BlockSpec((BLK, N_SPARSE), lambda i: (i, 0)),
          full(wd_pad.shape),
          full(wb.shape),
          full(W0.shape),
          full(b0.shape),
          full(W1.shape),
          full(b1.shape),
          full(W2.shape),
          full(b2.shape),
          full(w3r.shape),
          full(b3.shape),
      ],
      out_specs=pl.BlockSpec((BLK, 1), lambda i: (i, 0)),
      out_shape=jax.ShapeDtypeStruct((BATCH, 1), jnp.float32),
  )(h, xin, wval, wd_pad, wb, W0, b0, W1, b1, W2, b2, w3r, b3)


def kernel(inputs, embed_tables, wide_tables, w_dense, wide_b,
           W0, b0, W1, b1, W2, b2, W3, b3):
  sparse = inputs[:, N_DENSE:].astype(jnp.int32)
  offs = (jnp.arange(N_SPARSE, dtype=jnp.int32) * VOCAB)[None, :]
  flat_idx = (sparse + offs).reshape(-1)
  idx3d = flat_idx.reshape(NW * NCHUNK, NSEG, SEG)
  idx3d = jnp.pad(idx3d, ((0, 0), (0, NSEG_PAD - NSEG), (0, 0)))
  idx2d = idx3d.reshape(NW * NCHUNK * NSEG_PAD, SEG)
  emb_flat = embed_tables.reshape(N_SPARSE * VOCAB, EMBED_DIM)
  wide_flat = wide_tables.reshape(-1)

  h, wval = _sc_gather(emb_flat, wide_flat, idx2d)
  h = h.reshape(BATCH, N_SPARSE * EMBED_DIM)

  wd_pad = jnp.concatenate(
      [w_dense[:, 0], jnp.zeros((N_SPARSE,), jnp.float32)])[None, :]
  return _tc_mlp(h, inputs, wval.reshape(BATCH, N_SPARSE), wd_pad,
                 wide_b.reshape(1, 1), W0, b0[None, :], W1, b1[None, :],
                 W2, b2[None, :], W3.reshape(1, -1), b3.reshape(1, 1))


# speedup vs baseline: 9.0177x; 1.3085x over previous
import functools
import jax
import jax.numpy as jnp
from jax import lax
from jax.experimental import pallas as pl
from jax.experimental.pallas import tpu as pltpu
from jax.experimental.pallas import tpu_sc as plsc

N_DENSE = 13
N_SPARSE = 26
VOCAB = 100000
EMBED_DIM = 16
BATCH = 16384

VCH = 12544
NJ = 8
ROWS_F = VCH
TAIL = VOCAB - (NJ - 1) * VCH

NC = 2
NS = 16
LANES = 16
NW = NC * NS
B_PER_W = BATCH // NW
CH = 128
NCHUNK = B_PER_W // CH
ROWS = CH * N_SPARSE
SEG = 128
NSEG = ROWS // SEG
NSEG_PAD = 32


def _sc_gather(emb_flat, wide_flat, idx2d, idx2dw):
  mesh = plsc.VectorSubcoreMesh(core_axis_name="c", subcore_axis_name="s")

  @functools.partial(
      pl.kernel,
      mesh=mesh,
      compiler_params=pltpu.CompilerParams(use_tc_tiling_on_sc=False),
      out_type=[
          jax.ShapeDtypeStruct((BATCH * N_SPARSE, EMBED_DIM), jnp.float32),
          jax.ShapeDtypeStruct((BATCH * N_SPARSE,), jnp.float32),
      ],
      scratch_types=[
          pltpu.VMEM((NSEG_PAD, SEG), jnp.int32),
          pltpu.VMEM((NSEG_PAD, SEG), jnp.int32),
          pltpu.VMEM((ROWS, EMBED_DIM), jnp.float32),
          pltpu.VMEM((ROWS,), jnp.float32),
          pltpu.SemaphoreType.DMA,
          pltpu.SemaphoreType.DMA,
      ],
  )
  def k(emb_hbm, wide_hbm, idx_hbm, idxw_hbm, h_out, wval_out,
        idx_v, idxw_v, emb_v, wval_v, sem_e, sem_w):
    wid = lax.axis_index("s") * NC + lax.axis_index("c")

    def chunk_body(c, carry):
      seg0 = (wid * NCHUNK + c) * NSEG_PAD
      pltpu.sync_copy(idx_hbm.at[pl.ds(seg0, NSEG_PAD)], idx_v)
      pltpu.sync_copy(idxw_hbm.at[pl.ds(seg0, NSEG_PAD)], idxw_v)
      cps = []
      for j in range(NSEG):
        cps.append(pltpu.async_copy(
            emb_hbm.at[idx_v.at[j]], emb_v.at[pl.ds(j * SEG, SEG)], sem_e))
        cps.append(pltpu.async_copy(
            wide_hbm.at[idxw_v.at[j]], wval_v.at[pl.ds(j * SEG, SEG)], sem_w))
      for cp in cps:
        cp.wait()
      base = (wid * NCHUNK + c) * ROWS
      pltpu.sync_copy(emb_v, h_out.at[pl.ds(base, ROWS)])
      pltpu.sync_copy(wval_v, wval_out.at[pl.ds(base, ROWS)])
      return carry

    lax.fori_loop(0, NCHUNK, chunk_body, 0)

  return k(emb_flat, wide_flat, idx2d, idx2dw)


def _tc_detile(emb_t):
  def body(in_ref, out_ref):
    j = pl.program_id(1)
    t = in_ref[0].T
    for jj in range(NJ):
      @pl.when(j == jj)
      def _():
        out_ref[:, jj * EMBED_DIM:(jj + 1) * EMBED_DIM] = t

  return pl.pallas_call(
      body,
      grid=(N_SPARSE, NJ),
      in_specs=[pl.BlockSpec((1, EMBED_DIM, VCH), lambda f, j: (f, 0, j))],
      out_specs=pl.BlockSpec((ROWS_F, 8 * EMBED_DIM), lambda f, j: (f, 0)),
      out_shape=jax.ShapeDtypeStruct(
          (N_SPARSE * ROWS_F, 8 * EMBED_DIM), jnp.float32),
  )(emb_t)


def _tc_mlp(h, xin, wval, wd_pad, wb, W0, b0, W1, b1, W2, b2, w3r, b3):
  BLK = 2048
  grid = (BATCH // BLK,)

  def body(h_ref, x_ref, wv_ref, wd_ref, wb_ref,
           W0_ref, b0_ref, W1_ref, b1_ref, W2_ref, b2_ref, w3_ref, b3_ref,
           o_ref):
    a = jnp.dot(h_ref[...], W0_ref[...], preferred_element_type=jnp.float32)
    a = jnp.maximum(a + b0_ref[...], 0.0)
    a = jnp.dot(a, W1_ref[...], preferred_element_type=jnp.float32)
    a = jnp.maximum(a + b1_ref[...], 0.0)
    a = jnp.dot(a, W2_ref[...], preferred_element_type=jnp.float32)
    a = jnp.maximum(a + b2_ref[...], 0.0)
    deep = jnp.sum(a * w3_ref[...], axis=1, keepdims=True) + b3_ref[...]
    dense = jnp.sum(x_ref[...] * wd_ref[...], axis=1, keepdims=True)
    wsum = jnp.sum(wv_ref[...], axis=1, keepdims=True)
    wide = dense + wsum + wb_ref[...]
    o_ref[...] = jax.nn.sigmoid(0.5 * (wide + deep))

  full = lambda shape: pl.BlockSpec(shape, lambda i: (0, 0))
  return pl.pallas_call(
      body,
      grid=grid,
      in_specs=[
          pl.BlockSpec((BLK, N_SPARSE * EMBED_DIM), lambda i: (i, 0)),
          pl.BlockSpec((BLK, N_DENSE + N_SPARSE), lambda i: (i, 0)),
          pl.BlockSpec((BLK, N_SPARSE), lambda i: (i, 0)),
          full(wd_pad.shape),
          full(wb.shape),
          full(W0.shape),
          full(b0.shape),
          full(W1.shape),
          full(b1.shape),
          full(W2.shape),
          full(b2.shape),
          full(w3r.shape),
          full(b3.shape),
      ],
      out_specs=pl.BlockSpec((BLK, 1), lambda i: (i, 0)),
      out_shape=jax.ShapeDtypeStruct((BATCH, 1), jnp.float32),
  )(h, xin, wval, wd_pad, wb, W0, b0, W1, b1, W2, b2, w3r, b3)


def kernel(inputs, embed_tables, wide_tables, w_dense, wide_b,
           W0, b0, W1, b1, W2, b2, W3, b3):
  sparse = inputs[:, N_DENSE:].astype(jnp.int32)
  perm_v = (sparse % VCH) * 8 + sparse // VCH
  offs = (jnp.arange(N_SPARSE, dtype=jnp.int32) * (ROWS_F * 8))[None, :]
  offsw = (jnp.arange(N_SPARSE, dtype=jnp.int32) * VOCAB)[None, :]

  def pack(flat):
    i3 = flat.reshape(NW * NCHUNK, NSEG, SEG)
    i3 = jnp.pad(i3, ((0, 0), (0, NSEG_PAD - NSEG), (0, 0)))
    return i3.reshape(NW * NCHUNK * NSEG_PAD, SEG)

  idx2d = pack((perm_v + offs).reshape(-1))
  idx2dw = pack((sparse + offsw).reshape(-1))
  emb_t = jnp.transpose(embed_tables, (0, 2, 1))
  emb_flat = _tc_detile(emb_t).reshape(N_SPARSE * ROWS_F * 8, EMBED_DIM)
  wide_flat = wide_tables.reshape(-1)

  h, wval = _sc_gather(emb_flat, wide_flat, idx2d, idx2dw)
  h = h.reshape(BATCH, N_SPARSE * EMBED_DIM)

  wd_pad = jnp.concatenate(
      [w_dense[:, 0], jnp.zeros((N_SPARSE,), jnp.float32)])[None, :]
  return _tc_mlp(h, inputs, wval.reshape(BATCH, N_SPARSE), wd_pad,
                 wide_b.reshape(1, 1), W0, b0[None, :], W1, b1[None, :],
                 W2, b2[None, :], W3.reshape(1, -1), b3.reshape(1, 1))


# speedup vs baseline: 20.2725x; 2.2481x over previous
import functools
import jax
import jax.numpy as jnp
from jax import lax
from jax.experimental import pallas as pl
from jax.experimental.pallas import tpu as pltpu
from jax.experimental.pallas import tpu_sc as plsc

N_DENSE = 13
N_SPARSE = 26
VOCAB = 100000
EMBED_DIM = 16
BATCH = 16384

FPB = 8
FB = 4
VC2 = 2048
NJ2 = 49

NC = 2
NS = 16
LANES = 16
NW = NC * NS
B_PER_W = BATCH // NW
CH = 128
NCHUNK = B_PER_W // CH
ROWS = CH * N_SPARSE
SEG = 128
NSEG = ROWS // SEG
NSEG_PAD = 32


def _sc_gather(emb_flat, wide_flat, idx2d, idx2dw):
  mesh = plsc.VectorSubcoreMesh(core_axis_name="c", subcore_axis_name="s")

  @functools.partial(
      pl.kernel,
      mesh=mesh,
      compiler_params=pltpu.CompilerParams(use_tc_tiling_on_sc=False),
      out_type=[
          jax.ShapeDtypeStruct((BATCH * N_SPARSE, EMBED_DIM), jnp.float32),
          jax.ShapeDtypeStruct((BATCH * N_SPARSE,), jnp.float32),
      ],
      scratch_types=[
          pltpu.VMEM((NSEG_PAD, SEG), jnp.int32),
          pltpu.VMEM((NSEG_PAD, SEG), jnp.int32),
          pltpu.VMEM((ROWS, EMBED_DIM), jnp.float32),
          pltpu.VMEM((ROWS,), jnp.float32),
          pltpu.SemaphoreType.DMA,
          pltpu.SemaphoreType.DMA,
      ],
  )
  def k(emb_hbm, wide_hbm, idx_hbm, idxw_hbm, h_out, wval_out,
        idx_v, idxw_v, emb_v, wval_v, sem_e, sem_w):
    wid = lax.axis_index("s") * NC + lax.axis_index("c")

    def chunk_body(c, carry):
      seg0 = (wid * NCHUNK + c) * NSEG_PAD
      pltpu.sync_copy(idx_hbm.at[pl.ds(seg0, NSEG_PAD)], idx_v)
      pltpu.sync_copy(idxw_hbm.at[pl.ds(seg0, NSEG_PAD)], idxw_v)
      cps = []
      for j in range(NSEG):
        cps.append(pltpu.async_copy(
            emb_hbm.at[idx_v.at[j]], emb_v.at[pl.ds(j * SEG, SEG)], sem_e))
        cps.append(pltpu.async_copy(
            wide_hbm.at[idxw_v.at[j]], wval_v.at[pl.ds(j * SEG, SEG)], sem_w))
      for cp in cps:
        cp.wait()
      base = (wid * NCHUNK + c) * ROWS
      pltpu.sync_copy(emb_v, h_out.at[pl.ds(base, ROWS)])
      pltpu.sync_copy(wval_v, wval_out.at[pl.ds(base, ROWS)])
      return carry

    lax.fori_loop(0, NCHUNK, chunk_body, 0)

  return k(emb_flat, wide_flat, idx2d, idx2dw)


def _tc_detile(a2):
  def body(in_ref, out_ref):
    out_ref[...] = in_ref[...].T

  return pl.pallas_call(
      body,
      grid=(FB, NJ2),
      in_specs=[pl.BlockSpec((FPB * EMBED_DIM, VC2), lambda fb, j: (fb, j))],
      out_specs=pl.BlockSpec((VC2, FPB * EMBED_DIM),
                             lambda fb, j: (fb * NJ2 + j, 0)),
      out_shape=jax.ShapeDtypeStruct(
          (FB * NJ2 * VC2, FPB * EMBED_DIM), jnp.float32),
  )(a2)


def _tc_mlp(h, xin, wval, wd_pad, wb, W0, b0, W1, b1, W2, b2, w3r, b3):
  BLK = 2048
  grid = (BATCH // BLK,)

  def body(h_ref, x_ref, wv_ref, wd_ref, wb_ref,
           W0_ref, b0_ref, W1_ref, b1_ref, W2_ref, b2_ref, w3_ref, b3_ref,
           o_ref):
    a = jnp.dot(h_ref[...], W0_ref[...], preferred_element_type=jnp.float32)
    a = jnp.maximum(a + b0_ref[...], 0.0)
    a = jnp.dot(a, W1_ref[...], preferred_element_type=jnp.float32)
    a = jnp.maximum(a + b1_ref[...], 0.0)
    a = jnp.dot(a, W2_ref[...], preferred_element_type=jnp.float32)
    a = jnp.maximum(a + b2_ref[...], 0.0)
    deep = jnp.sum(a * w3_ref[...], axis=1, keepdims=True) + b3_ref[...]
    dense = jnp.sum(x_ref[...] * wd_ref[...], axis=1, keepdims=True)
    wsum = jnp.sum(wv_ref[...], axis=1, keepdims=True)
    wide = dense + wsum + wb_ref[...]
    o_ref[...] = jax.nn.sigmoid(0.5 * (wide + deep))

  full = lambda shape: pl.BlockSpec(shape, lambda i: (0, 0))
  return pl.pallas_call(
      body,
      grid=grid,
      in_specs=[
          pl.BlockSpec((BLK, N_SPARSE * EMBED_DIM), lambda i: (i, 0)),
          pl.BlockSpec((BLK, N_DENSE + N_SPARSE), lambda i: (i, 0)),
          pl.BlockSpec((BLK, N_SPARSE), lambda i: (i, 0)),
          full(wd_pad.shape),
          full(wb.shape),
          full(W0.shape),
          full(b0.shape),
          full(W1.shape),
          full(b1.shape),
          full(W2.shape),
          full(b2.shape),
          full(w3r.shape),
          full(b3.shape),
      ],
      out_specs=pl.BlockSpec((BLK, 1), lambda i: (i, 0)),
      out_shape=jax.ShapeDtypeStruct((BATCH, 1), jnp.float32),
  )(h, xin, wval, wd_pad, wb, W0, b0, W1, b1, W2, b2, w3r, b3)


def kernel(inputs, embed_tables, wide_tables, w_dense, wide_b,
           W0, b0, W1, b1, W2, b2, W3, b3):
  sparse = inputs[:, N_DENSE:].astype(jnp.int32)
  perm_v = (sparse // VC2) * (VC2 * 8) + (sparse % VC2) * 8
  farange = jnp.arange(N_SPARSE, dtype=jnp.int32)
  offs = ((farange // FPB) * (NJ2 * VC2 * 8) + farange % FPB)[None, :]
  offsw = (farange * VOCAB)[None, :]

  def pack(flat):
    i3 = flat.reshape(NW * NCHUNK, NSEG, SEG)
    i3 = jnp.pad(i3, ((0, 0), (0, NSEG_PAD - NSEG), (0, 0)))
    return i3.reshape(NW * NCHUNK * NSEG_PAD, SEG)

  idx2d = pack((perm_v + offs).reshape(-1))
  idx2dw = pack((sparse + offsw).reshape(-1))
  a2 = jnp.transpose(embed_tables, (0, 2, 1)).reshape(
      N_SPARSE * EMBED_DIM, VOCAB)
  emb_flat = _tc_detile(a2).reshape(FB * NJ2 * VC2 * 8, EMBED_DIM)
  wide_flat = wide_tables.reshape(-1)

  h, wval = _sc_gather(emb_flat, wide_flat, idx2d, idx2dw)
  h = h.reshape(BATCH, N_SPARSE * EMBED_DIM)

  wd_pad = jnp.concatenate(
      [w_dense[:, 0], jnp.zeros((N_SPARSE,), jnp.float32)])[None, :]
  return _tc_mlp(h, inputs, wval.reshape(BATCH, N_SPARSE), wd_pad,
                 wide_b.reshape(1, 1), W0, b0[None, :], W1, b1[None, :],
                 W2, b2[None, :], W3.reshape(1, -1), b3.reshape(1, 1))


# speedup vs baseline: 21.7689x; 1.0738x over previous
import functools
import jax
import jax.numpy as jnp
from jax import lax
from jax.experimental import pallas as pl
from jax.experimental.pallas import tpu as pltpu
from jax.experimental.pallas import tpu_sc as plsc

N_DENSE = 13
N_SPARSE = 26
VOCAB = 100000
EMBED_DIM = 16
BATCH = 16384

FPB = 8
FB = 4
VC2 = 2048
NJ2 = 49

NC = 2
NS = 16
LANES = 16
NW = NC * NS
B_PER_W = BATCH // NW
CH = 128
NCHUNK = B_PER_W // CH
ROWS = CH * N_SPARSE
SEG = 128
NSEG = ROWS // SEG
NSEG_PAD = 32


_MESH = plsc.VectorSubcoreMesh(core_axis_name="c", subcore_axis_name="s")
_SC_PARAMS = pltpu.CompilerParams(use_tc_tiling_on_sc=False)
WSEG = B_PER_W * N_SPARSE // SEG


def _sc_gather_emb(emb_flat, idx2d):
  @functools.partial(
      pl.kernel,
      mesh=_MESH,
      compiler_params=_SC_PARAMS,
      out_type=jax.ShapeDtypeStruct((BATCH * N_SPARSE, EMBED_DIM),
                                    jnp.float32),
      scratch_types=[
          pltpu.VMEM((NSEG_PAD, SEG), jnp.int32),
          pltpu.VMEM((ROWS, EMBED_DIM), jnp.float32),
          pltpu.SemaphoreType.DMA,
      ],
  )
  def k(emb_hbm, idx_hbm, h_out, idx_v, emb_v, sem_e):
    wid = lax.axis_index("s") * NC + lax.axis_index("c")

    def chunk_body(c, carry):
      seg0 = (wid * NCHUNK + c) * NSEG_PAD
      pltpu.sync_copy(idx_hbm.at[pl.ds(seg0, NSEG_PAD)], idx_v)
      cps = []
      for j in range(NSEG):
        cps.append(pltpu.async_copy(
            emb_hbm.at[idx_v.at[j]], emb_v.at[pl.ds(j * SEG, SEG)], sem_e))
      for cp in cps:
        cp.wait()
      base = (wid * NCHUNK + c) * ROWS
      pltpu.sync_copy(emb_v, h_out.at[pl.ds(base, ROWS)])
      return carry

    lax.fori_loop(0, NCHUNK, chunk_body, 0)

  return k(emb_flat, idx2d)


def _sc_gather_wide(wide_flat, idx2dw):
  @functools.partial(
      pl.kernel,
      mesh=_MESH,
      compiler_params=_SC_PARAMS,
      out_type=jax.ShapeDtypeStruct((BATCH * N_SPARSE,), jnp.float32),
      scratch_types=[
          pltpu.VMEM((WSEG, SEG), jnp.int32),
          pltpu.VMEM((WSEG * SEG,), jnp.float32),
          pltpu.SemaphoreType.DMA,
      ],
  )
  def k(wide_hbm, idxw_hbm, wval_out, idxw_v, wval_v, sem_w):
    wid = lax.axis_index("s") * NC + lax.axis_index("c")
    pltpu.sync_copy(idxw_hbm.at[pl.ds(wid * WSEG, WSEG)], idxw_v)
    cps = []
    for j in range(WSEG):
      cps.append(pltpu.async_copy(
          wide_hbm.at[idxw_v.at[j]], wval_v.at[pl.ds(j * SEG, SEG)], sem_w))
    for cp in cps:
      cp.wait()
    pltpu.sync_copy(wval_v, wval_out.at[pl.ds(wid * WSEG * SEG, WSEG * SEG)])

  return k(wide_flat, idx2dw)


def _tc_detile(a2):
  def body(in_ref, out_ref):
    out_ref[...] = in_ref[...].T

  return pl.pallas_call(
      body,
      grid=(FB, NJ2),
      in_specs=[pl.BlockSpec((FPB * EMBED_DIM, VC2), lambda fb, j: (fb, j))],
      out_specs=pl.BlockSpec((VC2, FPB * EMBED_DIM),
                             lambda fb, j: (fb * NJ2 + j, 0)),
      out_shape=jax.ShapeDtypeStruct(
          (FB * NJ2 * VC2, FPB * EMBED_DIM), jnp.float32),
  )(a2)


def _tc_mlp(h, xin, wval, wd_pad, wb, W0, b0, W1, b1, W2, b2, w3r, b3):
  BLK = 2048
  grid = (BATCH // BLK,)

  def body(h_ref, x_ref, wv_ref, wd_ref, wb_ref,
           W0_ref, b0_ref, W1_ref, b1_ref, W2_ref, b2_ref, w3_ref, b3_ref,
           o_ref):
    a = jnp.dot(h_ref[...], W0_ref[...], preferred_element_type=jnp.float32)
    a = jnp.maximum(a + b0_ref[...], 0.0)
    a = jnp.dot(a, W1_ref[...], preferred_element_type=jnp.float32)
    a = jnp.maximum(a + b1_ref[...], 0.0)
    a = jnp.dot(a, W2_ref[...], preferred_element_type=jnp.float32)
    a = jnp.maximum(a + b2_ref[...], 0.0)
    deep = jnp.sum(a * w3_ref[...], axis=1, keepdims=True) + b3_ref[...]
    dense = jnp.sum(x_ref[...] * wd_ref[...], axis=1, keepdims=True)
    wsum = jnp.sum(wv_ref[...], axis=1, keepdims=True)
    wide = dense + wsum + wb_ref[...]
    o_ref[...] = jax.nn.sigmoid(0.5 * (wide + deep))

  full = lambda shape: pl.BlockSpec(shape, lambda i: (0, 0))
  return pl.pallas_call(
      body,
      grid=grid,
      in_specs=[
          pl.BlockSpec((BLK, N_SPARSE * EMBED_DIM), lambda i: (i, 0)),
          pl.BlockSpec((BLK, N_DENSE + N_SPARSE), lambda i: (i, 0)),
          pl.BlockSpec((BLK, N_SPARSE), lambda i: (i, 0)),
          full(wd_pad.shape),
          full(wb.shape),
          full(W0.shape),
          full(b0.shape),
          full(W1.shape),
          full(b1.shape),
          full(W2.shape),
          full(b2.shape),
          full(w3r.shape),
          full(b3.shape),
      ],
      out_specs=pl.BlockSpec((BLK, 1), lambda i: (i, 0)),
      out_shape=jax.ShapeDtypeStruct((BATCH, 1), jnp.float32),
  )(h, xin, wval, wd_pad, wb, W0, b0, W1, b1, W2, b2, w3r, b3)


def kernel(inputs, embed_tables, wide_tables, w_dense, wide_b,
           W0, b0, W1, b1, W2, b2, W3, b3):
  sparse = inputs[:, N_DENSE:].astype(jnp.int32)
  perm_v = (sparse // VC2) * (VC2 * 8) + (sparse % VC2) * 8
  farange = jnp.arange(N_SPARSE, dtype=jnp.int32)
  offs = ((farange // FPB) * (NJ2 * VC2 * 8) + farange % FPB)[None, :]
  offsw = (farange * VOCAB)[None, :]

  def pack(flat):
    i3 = flat.reshape(NW * NCHUNK, NSEG, SEG)
    i3 = jnp.pad(i3, ((0, 0), (0, NSEG_PAD - NSEG), (0, 0)))
    return i3.reshape(NW * NCHUNK * NSEG_PAD, SEG)

  idx2d = pack((perm_v + offs).reshape(-1))
  idx2dw = (sparse + offsw).reshape(NW * WSEG, SEG)
  a2 = jnp.transpose(embed_tables, (0, 2, 1)).reshape(
      N_SPARSE * EMBED_DIM, VOCAB)
  emb_flat = _tc_detile(a2).reshape(FB * NJ2 * VC2 * 8, EMBED_DIM)
  wide_flat = wide_tables.reshape(-1)

  wval = _sc_gather_wide(wide_flat, idx2dw)
  h = _sc_gather_emb(emb_flat, idx2d)
  h = h.reshape(BATCH, N_SPARSE * EMBED_DIM)

  wd_pad = jnp.concatenate(
      [w_dense[:, 0], jnp.zeros((N_SPARSE,), jnp.float32)])[None, :]
  return _tc_mlp(h, inputs, wval.reshape(BATCH, N_SPARSE), wd_pad,
                 wide_b.reshape(1, 1), W0, b0[None, :], W1, b1[None, :],
                 W2, b2[None, :], W3.reshape(1, -1), b3.reshape(1, 1))
